# baseline (speedup 1.0000x reference)
import functools

import jax
import jax.numpy as jnp
from jax import lax
from jax.experimental import pallas as pl
from jax.experimental.pallas import tpu as pltpu
from jax.experimental.pallas import tpu_sc as plsc

GS = 52
SPACING = 2.0
NUM_CAMS = 12
JOINTS = 12
H, W = 512, 640
HW = H * W
V = GS * GS * GS
D = 16

NC, NS, L = 2, 16, 16
NW = NC * NS
CH = 896
NCH = 5
PW = CH * NCH
VPAD = NW * PW
NSEG = CH // 128


@functools.partial(
    pl.kernel,
    out_type=jax.ShapeDtypeStruct((VPAD, D), jnp.float32),
    mesh=plsc.VectorSubcoreMesh(core_axis_name="c", subcore_axis_name="s"),
    compiler_params=pltpu.CompilerParams(use_tc_tiling_on_sc=False),
    scratch_types=[
        pltpu.VMEM((NUM_CAMS * CH,), jnp.int32),
        pltpu.VMEM((CH, D), jnp.float32),
        pltpu.VMEM((CH, D), jnp.float32),
        pltpu.VMEM((CH, D), jnp.float32),
        pltpu.SemaphoreType.DMA,
        pltpu.SemaphoreType.DMA,
    ],
)
def _sc_gather_mean(hmT, idx_hbm, sums, idx_v, buf_a, buf_b, acc_v,
                    sem_a, sem_b):
    cid = lax.axis_index("c")
    sid = lax.axis_index("s")
    wid = sid * NC + cid

    def chunk_body(k, carry):
        cb = wid * PW + k * CH
        for c in range(NUM_CAMS):
            pltpu.sync_copy(idx_hbm.at[pl.ds(c * VPAD + cb, CH)],
                            idx_v.at[pl.ds(c * CH, CH)])

        def fire(c):
            buf = buf_a if c % 2 == 0 else buf_b
            sem = sem_a if c % 2 == 0 else sem_b
            handles = []
            for s in range(NSEG):
                handles.append(pltpu.async_copy(
                    hmT.at[idx_v.at[pl.ds(c * CH + s * 128, 128)]],
                    buf.at[pl.ds(s * 128, 128), :],
                    sem))
            return handles

        handles = fire(0)
        for c in range(NUM_CAMS):
            nxt = fire(c + 1) if c + 1 < NUM_CAMS else None
            for h in handles:
                h.wait()
            handles = nxt
            buf = buf_a if c % 2 == 0 else buf_b

            def acc_body(i, carry3, buf=buf, first=(c == 0)):
                row = buf[i, :]
                if first:
                    acc_v[i, :] = row
                else:
                    plsc.addupdate(acc_v.at[i, :], row)
                return carry3

            lax.fori_loop(0, CH, acc_body, 0)

        pltpu.sync_copy(acc_v, sums.at[pl.ds(cb, CH), :])
        return carry

    lax.fori_loop(0, NCH, chunk_body, 0)


def _grid():
    r = jnp.arange(GS, dtype=jnp.float32) - float(GS // 2)
    gx, gy, gz = jnp.meshgrid(r, r, r, indexing='ij')
    return jnp.stack([gx, gy, gz], axis=-1) * SPACING


def kernel(heatmaps, center, cameraMatrices):
    hm = heatmaps.reshape(NUM_CAMS, JOINTS, HW)
    hm = jnp.pad(hm, ((0, 0), (0, D - JOINTS), (0, 0)))
    hmT = jnp.transpose(hm, (0, 2, 1))
    hmT = hmT.reshape(NUM_CAMS * HW, D)

    grid = _grid()

    def per_batch_idx(c):
        g = grid + c
        ones = jnp.ones(g.shape[:3] + (1,), dtype=g.dtype)
        g4 = jnp.concatenate([g, ones], axis=-1)
        partial = jnp.einsum('xyzr,nrc->nxyzc', g4, cameraMatrices)
        u = jnp.clip(partial[..., 0] / partial[..., 2], 0.0, 1279.0)
        v = jnp.clip(partial[..., 1] / partial[..., 2], 0.0, 1023.0)
        idx = (v / 2.0).astype(jnp.int32) * 640 + (u / 2.0).astype(jnp.int32)
        return jax.lax.stop_gradient(idx)

    idx = jax.vmap(per_batch_idx)(center)[0]
    idx = idx.reshape(NUM_CAMS, V)
    idx = jnp.pad(idx, ((0, 0), (0, VPAD - V)))
    idx = idx + (jnp.arange(NUM_CAMS, dtype=jnp.int32) * HW)[:, None]
    idx = idx.reshape(NUM_CAMS * VPAD)

    sums = _sc_gather_mean(hmT, idx)

    out = sums[:V, :JOINTS].T * (1.0 / NUM_CAMS)
    return out.reshape(1, JOINTS, GS, GS, GS)


# speedup vs baseline: 1.0101x; 1.0101x over previous
import functools

import jax
import jax.numpy as jnp
from jax import lax
from jax.experimental import pallas as pl
from jax.experimental.pallas import tpu as pltpu
from jax.experimental.pallas import tpu_sc as plsc

GS = 52
SPACING = 2.0
NUM_CAMS = 12
JOINTS = 12
H, W = 512, 640
HW = H * W
V = GS * GS * GS
D = 16

NC, NS, L = 2, 16, 16
NW = NC * NS
CH = 896
NCH = 5
PW = CH * NCH
VPAD = NW * PW
NSEG = CH // 128


@functools.partial(
    pl.kernel,
    out_type=jax.ShapeDtypeStruct((VPAD, D), jnp.float32),
    mesh=plsc.VectorSubcoreMesh(core_axis_name="c", subcore_axis_name="s"),
    compiler_params=pltpu.CompilerParams(use_tc_tiling_on_sc=False),
    scratch_types=[
        pltpu.VMEM((NUM_CAMS * CH,), jnp.int32),
        pltpu.VMEM((CH, D), jnp.float32),
        pltpu.VMEM((CH, D), jnp.float32),
        pltpu.VMEM((CH, D), jnp.float32),
        pltpu.SemaphoreType.DMA,
        pltpu.SemaphoreType.DMA,
    ],
)
def _sc_gather_mean(hmT, idx_hbm, sums, idx_v, buf_a, buf_b, acc_v,
                    sem_a, sem_b):
    cid = lax.axis_index("c")
    sid = lax.axis_index("s")
    wid = sid * NC + cid

    def chunk_body(k, carry):
        cb = wid * PW + k * CH
        ih = [pltpu.async_copy(idx_hbm.at[pl.ds(c * VPAD + cb, CH)],
                               idx_v.at[pl.ds(c * CH, CH)], sem_a)
              for c in range(NUM_CAMS)]
        for h in ih:
            h.wait()

        def fire(c):
            buf = buf_a if c % 2 == 0 else buf_b
            sem = sem_a if c % 2 == 0 else sem_b
            handles = []
            for s in range(NSEG):
                handles.append(pltpu.async_copy(
                    hmT.at[idx_v.at[pl.ds(c * CH + s * 128, 128)]],
                    buf.at[pl.ds(s * 128, 128), :],
                    sem))
            return handles

        handles = fire(0)
        for c in range(NUM_CAMS):
            nxt = fire(c + 1) if c + 1 < NUM_CAMS else None
            for h in handles:
                h.wait()
            handles = nxt
            buf = buf_a if c % 2 == 0 else buf_b

            def acc_body(i, carry3, buf=buf, first=(c == 0)):
                row = buf[i, :]
                if first:
                    acc_v[i, :] = row
                else:
                    plsc.addupdate(acc_v.at[i, :], row)
                return carry3

            lax.fori_loop(0, CH, acc_body, 0, unroll=8)

        pltpu.sync_copy(acc_v, sums.at[pl.ds(cb, CH), :])
        return carry

    lax.fori_loop(0, NCH, chunk_body, 0)


def _grid():
    r = jnp.arange(GS, dtype=jnp.float32) - float(GS // 2)
    gx, gy, gz = jnp.meshgrid(r, r, r, indexing='ij')
    return jnp.stack([gx, gy, gz], axis=-1) * SPACING


def kernel(heatmaps, center, cameraMatrices):
    hm = heatmaps.reshape(NUM_CAMS, JOINTS, HW)
    hmT = jnp.transpose(hm, (0, 2, 1))
    hmT = jnp.pad(hmT, ((0, 0), (0, 0), (0, D - JOINTS)))
    hmT = hmT.reshape(NUM_CAMS * HW, D)

    grid = _grid()

    def per_batch_idx(c):
        g = grid + c
        ones = jnp.ones(g.shape[:3] + (1,), dtype=g.dtype)
        g4 = jnp.concatenate([g, ones], axis=-1)
        partial = jnp.einsum('xyzr,nrc->nxyzc', g4, cameraMatrices)
        u = jnp.clip(partial[..., 0] / partial[..., 2], 0.0, 1279.0)
        v = jnp.clip(partial[..., 1] / partial[..., 2], 0.0, 1023.0)
        idx = (v / 2.0).astype(jnp.int32) * 640 + (u / 2.0).astype(jnp.int32)
        return jax.lax.stop_gradient(idx)

    idx = jax.vmap(per_batch_idx)(center)[0]
    idx = idx.reshape(NUM_CAMS, V)
    idx = jnp.pad(idx, ((0, 0), (0, VPAD - V)))
    idx = idx + (jnp.arange(NUM_CAMS, dtype=jnp.int32) * HW)[:, None]
    idx = idx.reshape(NUM_CAMS * VPAD)

    sums = _sc_gather_mean(hmT, idx)

    out = sums[:V, :JOINTS].T * (1.0 / NUM_CAMS)
    return out.reshape(1, JOINTS, GS, GS, GS)
